# baseline (speedup 1.0000x reference)
import functools

import jax
import jax.numpy as jnp
from jax import lax
from jax.experimental import pallas as pl
from jax.experimental.pallas import tpu as pltpu
from jax.experimental.pallas import tpu_sc as plsc

_G = 32
_NOBJ = 50
_ANCHOR = 2.5
_EMPTY_WEIGHT = 0.5
_LANES = 16
_ROW = 128


def _pad_kernel(in_ref, out_ref):
    out_ref[:, :, :, 0:68] = in_ref[...]


def _make_sc_gather(B):
    info = plsc.get_sparse_core_info()
    nc, ns = info.num_cores, info.num_subcores
    nw = nc * ns
    total = B * _NOBJ
    per = total // nw
    assert per * nw == total and per % _LANES == 0

    mesh = plsc.VectorSubcoreMesh(core_axis_name="c", subcore_axis_name="s")

    @functools.partial(
        pl.kernel,
        mesh=mesh,
        out_type=jax.ShapeDtypeStruct((total, _ROW), jnp.float32),
        scratch_types=[
            pltpu.VMEM((per,), jnp.float32),
            pltpu.VMEM((per,), jnp.float32),
            pltpu.VMEM((per, _ROW), jnp.float32),
            pltpu.SemaphoreType.DMA,
        ],
    )
    def sc_gather(xcol_hbm, ycol_hbm, table_hbm, w_hbm, xv, yv, rv, sem):
        wid = lax.axis_index("s") * nc + lax.axis_index("c")
        base = wid * per
        pltpu.sync_copy(xcol_hbm.at[pl.ds(base, per)], xv)
        pltpu.sync_copy(ycol_hbm.at[pl.ds(base, per)], yv)

        copies = []
        for c in range(per // _LANES):
            lo = c * _LANES + lax.broadcasted_iota(jnp.int32, (_LANES,), 0)
            x = xv[pl.ds(c * _LANES, _LANES)]
            y = yv[pl.ds(c * _LANES, _LANES)]
            gi = jnp.clip((x * float(_G)).astype(jnp.int32), 0, _G - 1)
            gj = jnp.clip((y * float(_G)).astype(jnp.int32), 0, _G - 1)
            b = lax.div(base + lo, _NOBJ)
            r = b * (_G * _G) + gj * _G + gi
            sl = pl.ds(c * _LANES, _LANES)
            copies.append(pltpu.async_copy(table_hbm.at[r], rv.at[sl], sem))
        for cp in copies:
            cp.wait()
        pltpu.sync_copy(rv, w_hbm.at[pl.ds(base, per)])

    return sc_gather


def _loss_kernel(nblocks, w_ref, tb_ref, txt_ref, tyt_ref, tk_ref,
                 acc_ref, loss_ref):
    tb = tb_ref[...]
    tk = tk_ref[...]
    nb = tb.shape[0]

    tx_full = tb[:, :, 0:1] * float(_G)
    ty_full = tb[:, :, 1:2] * float(_G)
    gi = jnp.clip(tx_full.astype(jnp.int32), 0, _G - 1)
    gj = jnp.clip(ty_full.astype(jnp.int32), 0, _G - 1)
    cell = gj * _G + gi

    g = w_ref[...].reshape(nb, _NOBJ, _ROW)

    txt = txt_ref[...].reshape(nb, 1, _NOBJ) * float(_G)
    tyt = tyt_ref[...].reshape(nb, 1, _NOBJ) * float(_G)
    gi_t = jnp.clip(txt.astype(jnp.int32), 0, _G - 1)
    gj_t = jnp.clip(tyt.astype(jnp.int32), 0, _G - 1)
    cell_t = gj_t * _G + gi_t

    eq = cell == cell_t
    row_i = lax.broadcasted_iota(jnp.int32, (nb, _NOBJ, _NOBJ), 1)
    col_j = lax.broadcasted_iota(jnp.int32, (nb, _NOBJ, _NOBJ), 2)
    later_same = jnp.logical_and(eq, col_j > row_i)
    loses = jnp.any(later_same, axis=2, keepdims=True)
    winner = jnp.logical_not(loses).astype(jnp.float32)

    x = g[:, :, 0:1]
    y = g[:, :, 1:2]
    w = g[:, :, 2:3]
    h = g[:, :, 3:4]
    conf = g[:, :, 4:5]
    keyp = g[:, :, 5:68]

    fx = tx_full - jnp.floor(tx_full)
    fy = ty_full - jnp.floor(ty_full)
    tw = tb[:, :, 3:4] * (float(_G) / _ANCHOR)
    th = tb[:, :, 4:5] * (float(_G) / _ANCHOR)

    sx = jnp.sum(winner * (x - fx) ** 2)
    sy = jnp.sum(winner * (y - fy) ** 2)
    sw = jnp.sum(winner * (w - tw) ** 2)
    sh = jnp.sum(winner * (h - th) ** 2)

    logp = jnp.maximum(jnp.log(conf), -100.0)
    sconf = jnp.sum(winner * (_EMPTY_WEIGHT * (-logp)))

    skp = jnp.sum(winner * jnp.abs(keyp - tk))

    b1x1, b1x2 = x - w * 0.5, x + w * 0.5
    b1y1, b1y2 = y - h * 0.5, y + h * 0.5
    b2x1, b2x2 = fx - tw * 0.5, fx + tw * 0.5
    b2y1, b2y2 = fy - th * 0.5, fy + th * 0.5
    iw = jnp.maximum(jnp.minimum(b1x2, b2x2) - jnp.maximum(b1x1, b2x1), 0.0)
    ih = jnp.maximum(jnp.minimum(b1y2, b2y2) - jnp.maximum(b1y1, b2y1), 0.0)
    inter = iw * ih
    union = w * h + tw * th - inter + 1e-16
    iou = inter / union
    siou = jnp.sum(winner * (1.0 - iou))

    n_here = jnp.sum(winner)

    vals = jnp.stack([sx, sy, sw, sh, sconf, skp, siou, n_here]).reshape(1, 8)

    @pl.when(pl.program_id(0) == 0)
    def _init():
        acc_ref[...] = jnp.zeros_like(acc_ref)

    acc_ref[...] += vals

    @pl.when(pl.program_id(0) == nblocks - 1)
    def _final():
        s = acc_ref[0]
        n = s[7]
        loss_boxes = (s[0] + s[1] + s[2] + s[3]) / n
        loss_keypoint = s[5] / n
        loss_conf = s[4] / (n * n)
        loss_iou = s[6] / n
        loss_ref[...] = jnp.stack(
            [loss_boxes, loss_keypoint, loss_conf, loss_iou]).reshape(1, 4)


@jax.jit
def kernel(outputs, target_boxes, target_keypoints):
    B, G = outputs.shape[0], outputs.shape[1]
    C = outputs.shape[3]
    BB = 4
    padded = pl.pallas_call(
        _pad_kernel,
        grid=(B // BB,),
        in_specs=[pl.BlockSpec((BB, G, G, C), lambda i: (i, 0, 0, 0))],
        out_specs=pl.BlockSpec((BB, G, G, _ROW), lambda i: (i, 0, 0, 0)),
        out_shape=jax.ShapeDtypeStruct((B, G, G, _ROW), jnp.float32),
    )(outputs)
    table = padded.reshape(B * G * G, _ROW)
    xcol = target_boxes[:, :, 0].reshape(B * _NOBJ)
    ycol = target_boxes[:, :, 1].reshape(B * _NOBJ)

    w = _make_sc_gather(B)(xcol, ycol, table)

    NB = 64
    nblocks = B // NB
    _, losses = pl.pallas_call(
        functools.partial(_loss_kernel, nblocks),
        grid=(nblocks,),
        in_specs=[
            pl.BlockSpec((NB * _NOBJ, _ROW), lambda i: (i, 0)),
            pl.BlockSpec((NB, _NOBJ, 6), lambda i: (i, 0, 0)),
            pl.BlockSpec((NB, _NOBJ), lambda i: (i, 0)),
            pl.BlockSpec((NB, _NOBJ), lambda i: (i, 0)),
            pl.BlockSpec((NB, _NOBJ, 63), lambda i: (i, 0, 0)),
        ],
        out_specs=[
            pl.BlockSpec((1, 8), lambda i: (0, 0)),
            pl.BlockSpec((1, 4), lambda i: (0, 0)),
        ],
        out_shape=[
            jax.ShapeDtypeStruct((1, 8), jnp.float32),
            jax.ShapeDtypeStruct((1, 4), jnp.float32),
        ],
    )(w, target_boxes, target_boxes[:, :, 0], target_boxes[:, :, 1],
      target_keypoints)

    return (losses[0, 0], losses[0, 1], losses[0, 2], losses[0, 3])


# speedup vs baseline: 1.1044x; 1.1044x over previous
import functools

import jax
import jax.numpy as jnp
from jax import lax
from jax.experimental import pallas as pl
from jax.experimental.pallas import tpu as pltpu
from jax.experimental.pallas import tpu_sc as plsc

_G = 32
_NOBJ = 50
_ANCHOR = 2.5
_EMPTY_WEIGHT = 0.5
_LANES = 16
_ROW = 128


def _pad_kernel(in_ref, out_ref):
    out_ref[:, :, :, 0:68] = in_ref[...]


def _make_sc_gather(B):
    info = plsc.get_sparse_core_info()
    nc, ns = info.num_cores, info.num_subcores
    nw = nc * ns
    total = B * _NOBJ
    per = total // nw
    assert per * nw == total and per % _LANES == 0

    mesh = plsc.VectorSubcoreMesh(core_axis_name="c", subcore_axis_name="s")

    @functools.partial(
        pl.kernel,
        mesh=mesh,
        out_type=jax.ShapeDtypeStruct((total, _ROW), jnp.float32),
        scratch_types=[
            pltpu.VMEM((per,), jnp.float32),
            pltpu.VMEM((per,), jnp.float32),
            pltpu.VMEM((per, _ROW), jnp.float32),
            pltpu.SemaphoreType.DMA,
        ],
    )
    def sc_gather(xcol_hbm, ycol_hbm, table_hbm, w_hbm, xv, yv, rv, sem):
        wid = lax.axis_index("s") * nc + lax.axis_index("c")
        base = wid * per
        pltpu.sync_copy(xcol_hbm.at[pl.ds(base, per)], xv)
        pltpu.sync_copy(ycol_hbm.at[pl.ds(base, per)], yv)

        copies = []
        for c in range(per // _LANES):
            lo = c * _LANES + lax.broadcasted_iota(jnp.int32, (_LANES,), 0)
            x = xv[pl.ds(c * _LANES, _LANES)]
            y = yv[pl.ds(c * _LANES, _LANES)]
            gi = jnp.clip((x * float(_G)).astype(jnp.int32), 0, _G - 1)
            gj = jnp.clip((y * float(_G)).astype(jnp.int32), 0, _G - 1)
            b = lax.div(base + lo, _NOBJ)
            r = b * (_G * _G) + gj * _G + gi
            sl = pl.ds(c * _LANES, _LANES)
            copies.append(pltpu.async_copy(table_hbm.at[r], rv.at[sl], sem))
        for cp in copies:
            cp.wait()
        pltpu.sync_copy(rv, w_hbm.at[pl.ds(base, per)])

    return sc_gather


def _loss_kernel(nblocks, w_ref, tb_ref, txt_ref, tyt_ref, tk_ref,
                 acc_ref, loss_ref):
    tb = tb_ref[...]
    tk = tk_ref[...]
    nb = tb.shape[0]

    tx_full = tb[:, :, 0:1] * float(_G)
    ty_full = tb[:, :, 1:2] * float(_G)
    gi = jnp.clip(tx_full.astype(jnp.int32), 0, _G - 1)
    gj = jnp.clip(ty_full.astype(jnp.int32), 0, _G - 1)
    cell = gj * _G + gi

    g = w_ref[...].reshape(nb, _NOBJ, _ROW)

    txt = txt_ref[...].reshape(nb, 1, _NOBJ) * float(_G)
    tyt = tyt_ref[...].reshape(nb, 1, _NOBJ) * float(_G)
    gi_t = jnp.clip(txt.astype(jnp.int32), 0, _G - 1)
    gj_t = jnp.clip(tyt.astype(jnp.int32), 0, _G - 1)
    cell_t = gj_t * _G + gi_t

    eq = cell == cell_t
    row_i = lax.broadcasted_iota(jnp.int32, (nb, _NOBJ, _NOBJ), 1)
    col_j = lax.broadcasted_iota(jnp.int32, (nb, _NOBJ, _NOBJ), 2)
    later_same = jnp.logical_and(eq, col_j > row_i)
    loses = jnp.any(later_same, axis=2, keepdims=True)
    winner = jnp.logical_not(loses).astype(jnp.float32)

    x = g[:, :, 0:1]
    y = g[:, :, 1:2]
    w = g[:, :, 2:3]
    h = g[:, :, 3:4]
    conf = g[:, :, 4:5]
    keyp = g[:, :, 5:68]

    fx = tx_full - jnp.floor(tx_full)
    fy = ty_full - jnp.floor(ty_full)
    tw = tb[:, :, 3:4] * (float(_G) / _ANCHOR)
    th = tb[:, :, 4:5] * (float(_G) / _ANCHOR)

    sx = jnp.sum(winner * (x - fx) ** 2)
    sy = jnp.sum(winner * (y - fy) ** 2)
    sw = jnp.sum(winner * (w - tw) ** 2)
    sh = jnp.sum(winner * (h - th) ** 2)

    logp = jnp.maximum(jnp.log(conf), -100.0)
    sconf = jnp.sum(winner * (_EMPTY_WEIGHT * (-logp)))

    skp = jnp.sum(winner * jnp.abs(keyp - tk))

    b1x1, b1x2 = x - w * 0.5, x + w * 0.5
    b1y1, b1y2 = y - h * 0.5, y + h * 0.5
    b2x1, b2x2 = fx - tw * 0.5, fx + tw * 0.5
    b2y1, b2y2 = fy - th * 0.5, fy + th * 0.5
    iw = jnp.maximum(jnp.minimum(b1x2, b2x2) - jnp.maximum(b1x1, b2x1), 0.0)
    ih = jnp.maximum(jnp.minimum(b1y2, b2y2) - jnp.maximum(b1y1, b2y1), 0.0)
    inter = iw * ih
    union = w * h + tw * th - inter + 1e-16
    iou = inter / union
    siou = jnp.sum(winner * (1.0 - iou))

    n_here = jnp.sum(winner)

    vals = jnp.stack([sx, sy, sw, sh, sconf, skp, siou, n_here]).reshape(1, 8)

    @pl.when(pl.program_id(0) == 0)
    def _init():
        acc_ref[...] = jnp.zeros_like(acc_ref)

    acc_ref[...] += vals

    @pl.when(pl.program_id(0) == nblocks - 1)
    def _final():
        s = acc_ref[0]
        n = s[7]
        loss_boxes = (s[0] + s[1] + s[2] + s[3]) / n
        loss_keypoint = s[5] / n
        loss_conf = s[4] / (n * n)
        loss_iou = s[6] / n
        loss_ref[...] = jnp.stack(
            [loss_boxes, loss_keypoint, loss_conf, loss_iou]).reshape(1, 4)


@jax.jit
def kernel(outputs, target_boxes, target_keypoints):
    B, G = outputs.shape[0], outputs.shape[1]
    C = outputs.shape[3]
    padded = jnp.pad(outputs, ((0, 0), (0, 0), (0, 0), (0, _ROW - C)))
    table = padded.reshape(B * G * G, _ROW)
    xcol = target_boxes[:, :, 0].reshape(B * _NOBJ)
    ycol = target_boxes[:, :, 1].reshape(B * _NOBJ)

    w = _make_sc_gather(B)(xcol, ycol, table)

    NB = 64
    nblocks = B // NB
    _, losses = pl.pallas_call(
        functools.partial(_loss_kernel, nblocks),
        grid=(nblocks,),
        in_specs=[
            pl.BlockSpec((NB * _NOBJ, _ROW), lambda i: (i, 0)),
            pl.BlockSpec((NB, _NOBJ, 6), lambda i: (i, 0, 0)),
            pl.BlockSpec((NB, _NOBJ), lambda i: (i, 0)),
            pl.BlockSpec((NB, _NOBJ), lambda i: (i, 0)),
            pl.BlockSpec((NB, _NOBJ, 63), lambda i: (i, 0, 0)),
        ],
        out_specs=[
            pl.BlockSpec((1, 8), lambda i: (0, 0)),
            pl.BlockSpec((1, 4), lambda i: (0, 0)),
        ],
        out_shape=[
            jax.ShapeDtypeStruct((1, 8), jnp.float32),
            jax.ShapeDtypeStruct((1, 4), jnp.float32),
        ],
    )(w, target_boxes, target_boxes[:, :, 0], target_boxes[:, :, 1],
      target_keypoints)

    return (losses[0, 0], losses[0, 1], losses[0, 2], losses[0, 3])
